# baseline (speedup 1.0000x reference)
import jax
import jax.numpy as jnp
from jax.experimental import pallas as pl

DEPTH = 12
N = 2 ** DEPTH - 1


def _dot(a, b):
    return jax.lax.dot_general(
        a, b, (((1,), (0,)), ((), ())),
        preferred_element_type=jnp.float32)


def _sig(x):
    return 0.5 * jnp.tanh(0.5 * x) + 0.5


def _tree_kernel(feat_ref, wiou_ref, biou_ref, uiou_ref, wf_ref, bf_ref,
                 uf_ref, h_ref, c_ref):
    B = feat_ref.shape[0] // N
    H = uf_ref.shape[0]
    CH = 2048

    for d in range(DEPTH - 1, -1, -1):
        n = 2 ** d
        rows = n * B
        r0 = (n - 1) * B
        cb = (2 * n - 1) * B
        for c0 in range(0, rows, CH):
            cr = min(CH, rows - c0)
            x = feat_ref[pl.ds(r0 + c0, cr), :]
            iou = _dot(x, wiou_ref[:, :]) + biou_ref[0, :]
            if d < DEPTH - 1:
                ch = h_ref[pl.ds(cb + 2 * c0, 2 * cr), :].astype(jnp.bfloat16)
                cc = c_ref[pl.ds(cb + 2 * c0, 2 * cr), :]
                ch3 = ch.reshape(cr // B, 2 * B, H)
                hs = (ch3[:, :B, :] + ch3[:, B:, :]).reshape(cr, H)
                cc3 = cc.reshape(cr // B, 2 * B, H)
                cs = (cc3[:, :B, :] + cc3[:, B:, :]).reshape(cr, H)
                iou = iou + _dot(hs, uiou_ref[:, :])
                fx = _dot(x, wf_ref[:, :]) + bf_ref[0, :]
                g = _dot(ch, uf_ref[:, :]) * cc
                g3 = g.reshape(cr // B, 2 * B, H)
                gs = (g3[:, :B, :] + g3[:, B:, :]).reshape(cr, H)
            i = _sig(iou[:, :H])
            o = _sig(iou[:, H:2 * H])
            u = jnp.tanh(iou[:, 2 * H:])
            c = i * u
            if d < DEPTH - 1:
                c = c + fx * cs + gs
            h = o * jnp.tanh(c)
            h_ref[pl.ds(r0 + c0, cr), :] = h
            c_ref[pl.ds(r0 + c0, cr), :] = c


def kernel(features, descendants, parents, W_iou, b_iou, U_iou, W_f, b_f,
           U_f):
    del descendants, parents
    B, Nn, D = features.shape
    H = U_f.shape[0]
    featT = jnp.transpose(features, (1, 0, 2)).reshape(Nn * B, D)
    featT = featT.astype(jnp.bfloat16)
    h_t, c_t = pl.pallas_call(
        _tree_kernel,
        out_shape=[jax.ShapeDtypeStruct((Nn * B, H), jnp.float32)] * 2,
    )(featT, W_iou.T.astype(jnp.bfloat16), b_iou.reshape(1, -1),
      U_iou.T.astype(jnp.bfloat16), W_f.T.astype(jnp.bfloat16),
      b_f.reshape(1, -1), U_f.T.astype(jnp.bfloat16))
    h = h_t.reshape(Nn, B, H).transpose(1, 0, 2)
    c = c_t.reshape(Nn, B, H).transpose(1, 0, 2)
    return (h, c)


# speedup vs baseline: 1.1632x; 1.1632x over previous
import jax
import jax.numpy as jnp
from jax.experimental import pallas as pl

DEPTH = 12
N = 2 ** DEPTH - 1


def _dot(a, b):
    return jax.lax.dot_general(
        a, b, (((1,), (0,)), ((), ())),
        preferred_element_type=jnp.float32)


def _sig(x):
    return 0.5 * jnp.tanh(0.5 * x) + 0.5


def _tree_kernel(feat_ref, wiou_ref, biou_ref, uiou_ref, wf_ref, bf_ref,
                 uf_ref, h_ref, c_ref):
    B = feat_ref.shape[0] // N
    H = uf_ref.shape[0]
    CH = 2048

    for d in range(DEPTH - 1, -1, -1):
        n = 2 ** d
        rows = n * B
        r0 = (n - 1) * B
        cb = (2 * n - 1) * B
        for c0 in range(0, rows, CH):
            cr = min(CH, rows - c0)
            x = feat_ref[pl.ds(r0 + c0, cr), :].astype(jnp.bfloat16)
            iou = _dot(x, wiou_ref[:, :]) + biou_ref[0, :]
            if d < DEPTH - 1:
                ch = h_ref[pl.ds(cb + 2 * c0, 2 * cr), :].astype(jnp.bfloat16)
                cc = c_ref[pl.ds(cb + 2 * c0, 2 * cr), :]
                ch3 = ch.reshape(cr // B, 2 * B, H)
                hs = (ch3[:, :B, :] + ch3[:, B:, :]).reshape(cr, H)
                cc3 = cc.reshape(cr // B, 2 * B, H)
                cs = (cc3[:, :B, :] + cc3[:, B:, :]).reshape(cr, H)
                iou = iou + _dot(hs, uiou_ref[:, :])
                fx = _dot(x, wf_ref[:, :]) + bf_ref[0, :]
                g = _dot(ch, uf_ref[:, :]) * cc
                g3 = g.reshape(cr // B, 2 * B, H)
                gs = (g3[:, :B, :] + g3[:, B:, :]).reshape(cr, H)
            i = _sig(iou[:, :H])
            o = _sig(iou[:, H:2 * H])
            u = jnp.tanh(iou[:, 2 * H:])
            c = i * u
            if d < DEPTH - 1:
                c = c + fx * cs + gs
            h = o * jnp.tanh(c)
            h_ref[pl.ds(r0 + c0, cr), :] = h
            c_ref[pl.ds(r0 + c0, cr), :] = c


def kernel(features, descendants, parents, W_iou, b_iou, U_iou, W_f, b_f,
           U_f):
    del descendants, parents
    B, Nn, D = features.shape
    H = U_f.shape[0]
    featT = jnp.transpose(features, (1, 0, 2)).reshape(Nn * B, D)
    h_t, c_t = pl.pallas_call(
        _tree_kernel,
        out_shape=[jax.ShapeDtypeStruct((Nn * B, H), jnp.float32)] * 2,
    )(featT, W_iou.T.astype(jnp.bfloat16), b_iou.reshape(1, -1),
      U_iou.T.astype(jnp.bfloat16), W_f.T.astype(jnp.bfloat16),
      b_f.reshape(1, -1), U_f.T.astype(jnp.bfloat16))
    h = h_t.reshape(Nn, B, H).transpose(1, 0, 2)
    c = c_t.reshape(Nn, B, H).transpose(1, 0, 2)
    return (h, c)
